# baseline (speedup 1.0000x reference)
import functools

import jax
import jax.numpy as jnp
from jax import lax
from jax.experimental import pallas as pl
from jax.experimental.pallas import tpu as pltpu
from jax.experimental.pallas import tpu_sc as plsc

N_ROWS = 100000
N_J = 64
N_C = 32
ROW_W = N_J * N_C
B_TOT = 16384
NC, NS, L = 2, 16, 16
NW = NC * NS
BPW = B_TOT // NW
CH = 16
NCHUNK = BPW // CH


def _sc_gather_kernel(a2, b32, c32):
    mesh = plsc.VectorSubcoreMesh(core_axis_name="core", subcore_axis_name="sub",
                                  num_cores=NC, num_subcores=NS)

    @functools.partial(
        pl.kernel,
        out_type=jax.ShapeDtypeStruct((B_TOT, N_J), jnp.float32),
        mesh=mesh,
        compiler_params=pltpu.CompilerParams(use_tc_tiling_on_sc=False,
                                             needs_layout_passes=False),
        scratch_types=[
            pltpu.VMEM((BPW,), jnp.int32),
            pltpu.VMEM((BPW,), jnp.int32),
            pltpu.VMEM((CH, ROW_W), jnp.float32),
            pltpu.VMEM((CH, ROW_W), jnp.float32),
            pltpu.VMEM((BPW, N_J), jnp.float32),
            pltpu.SemaphoreType.DMA,
            pltpu.SemaphoreType.DMA,
            pltpu.SemaphoreType.DMA,
        ],
    )
    def k(a_hbm, b_hbm, c_hbm, out_hbm, bidx, cidx, rows0, rows1, outb,
          sem0, sem1, sem_idx):
        wid = lax.axis_index("sub") * NC + lax.axis_index("core")
        base = wid * BPW
        cp_b = pltpu.async_copy(b_hbm.at[pl.ds(base, BPW)], bidx, sem_idx)
        cp_c = pltpu.async_copy(c_hbm.at[pl.ds(base, BPW)], cidx, sem_idx)
        cp_b.wait()
        cp_c.wait()

        lane = lax.iota(jnp.int32, L)
        bufs = (rows0, rows1)
        sems = (sem0, sem1)

        def fire(ci):
            return pltpu.async_copy(
                a_hbm.at[bidx.at[pl.ds(ci * CH, CH)]], bufs[ci % 2],
                sems[ci % 2])

        def extract(ci, rows):
            cvec = cidx[pl.ds(ci * CH, L)]
            orow = ci * CH + lane

            def jbody(j, col):
                vals = plsc.load_gather(rows, [lane, col])
                plsc.store_scatter(outb, [orow, jnp.full((L,), j, jnp.int32)],
                                   vals)
                return col + N_C

            lax.fori_loop(0, N_J, jbody, cvec, unroll=4)

        pending = {0: fire(0)}
        for ci in range(NCHUNK):
            if ci + 1 < NCHUNK:
                pending[ci + 1] = fire(ci + 1)
            pending.pop(ci).wait()
            extract(ci, bufs[ci % 2])

        pltpu.sync_copy(outb, out_hbm.at[pl.ds(base, BPW)])

    return k(a2, b32, c32)


def kernel(a, b, c):
    a2 = a.reshape(N_ROWS, ROW_W)
    b32 = b.astype(jnp.int32)
    c32 = c.astype(jnp.int32)
    return _sc_gather_kernel(a2, b32, c32)
